# speedup vs baseline: 1.9846x; 1.9846x over previous
import functools

import jax
import jax.numpy as jnp
from jax import lax
from jax.experimental import pallas as pl
from jax.experimental.pallas import tpu as pltpu
from jax.experimental.pallas import tpu_sc as plsc

D_MODEL = 1024
EMBED_LEN = 8192
NC = 2
NS = 16
NW = NC * NS
CHUNK = 32


def _pe_gather(positions_hbm, table_hbm, out_hbm, idx_v, rows_v, sem):
    n_total = out_hbm.shape[0]
    b_per_w = n_total // NW
    n_chunks = b_per_w // CHUNK

    wid = lax.axis_index("s") * NC + lax.axis_index("c")
    base = wid * b_per_w

    pltpu.sync_copy(positions_hbm.at[pl.ds(base, b_per_w)], idx_v)

    def chunk_body(i, carry):
        off = pl.multiple_of(i * CHUNK, CHUNK)
        idx_chunk = idx_v.at[pl.ds(off, CHUNK)]
        pltpu.async_copy(table_hbm.at[idx_chunk], rows_v, sem).wait()
        pltpu.sync_copy(rows_v, out_hbm.at[pl.ds(base + off, CHUNK)])
        return carry

    lax.fori_loop(0, n_chunks, chunk_body, 0)


@jax.jit
def _pe_lookup(positions_flat, pe_table):
    n_total = positions_flat.shape[0]
    mesh = plsc.VectorSubcoreMesh(core_axis_name="c", subcore_axis_name="s")
    k = pl.kernel(
        _pe_gather,
        out_type=jax.ShapeDtypeStruct((n_total, D_MODEL), jnp.float32),
        mesh=mesh,
        scratch_types=[
            pltpu.VMEM((n_total // NW,), jnp.int32),
            pltpu.VMEM((CHUNK, D_MODEL), jnp.float32),
            pltpu.SemaphoreType.DMA,
        ],
    )
    return k(positions_flat, pe_table)


def kernel(positions, pe_table):
    b, s = positions.shape
    out = _pe_lookup(positions.reshape(b * s), pe_table)
    return out.reshape(b, s, pe_table.shape[1])


# speedup vs baseline: 2.3814x; 1.1999x over previous
import functools

import jax
import jax.numpy as jnp
from jax import lax
from jax.experimental import pallas as pl
from jax.experimental.pallas import tpu as pltpu
from jax.experimental.pallas import tpu_sc as plsc

D_MODEL = 1024
EMBED_LEN = 8192
NC = 2
NS = 16
NW = NC * NS
CHUNK = 32


def _pe_gather(positions_hbm, table_hbm, out_hbm, idx_v, rows_a, rows_b,
               sem_a, sem_b):
    n_total = out_hbm.shape[0]
    b_per_w = n_total // NW
    n_pairs = b_per_w // (2 * CHUNK)

    wid = lax.axis_index("s") * NC + lax.axis_index("c")
    base = wid * b_per_w

    pltpu.sync_copy(positions_hbm.at[pl.ds(base, b_per_w)], idx_v)

    def gather(i, buf, sem):
        off = pl.multiple_of(i * CHUNK, CHUNK)
        return pltpu.async_copy(table_hbm.at[idx_v.at[pl.ds(off, CHUNK)]],
                                buf, sem)

    def store(i, buf):
        off = pl.multiple_of(i * CHUNK, CHUNK)
        pltpu.sync_copy(buf, out_hbm.at[pl.ds(base + off, CHUNK)])

    gather(0, rows_a, sem_a)

    def pair_body(p, carry):
        i0 = 2 * p
        gather(i0 + 1, rows_b, sem_b)
        pltpu.make_async_copy(table_hbm.at[idx_v.at[pl.ds(0, CHUNK)]],
                              rows_a, sem_a).wait()
        store(i0, rows_a)

        @pl.when(p + 1 < n_pairs)
        def _():
            gather(i0 + 2, rows_a, sem_a)

        pltpu.make_async_copy(table_hbm.at[idx_v.at[pl.ds(0, CHUNK)]],
                              rows_b, sem_b).wait()
        store(i0 + 1, rows_b)
        return carry

    lax.fori_loop(0, n_pairs, pair_body, 0)


@jax.jit
def _pe_lookup(positions_flat, pe_table):
    n_total = positions_flat.shape[0]
    mesh = plsc.VectorSubcoreMesh(core_axis_name="c", subcore_axis_name="s")
    k = pl.kernel(
        _pe_gather,
        out_type=jax.ShapeDtypeStruct((n_total, D_MODEL), jnp.float32),
        mesh=mesh,
        scratch_types=[
            pltpu.VMEM((n_total // NW,), jnp.int32),
            pltpu.VMEM((CHUNK, D_MODEL), jnp.float32),
            pltpu.VMEM((CHUNK, D_MODEL), jnp.float32),
            pltpu.SemaphoreType.DMA,
            pltpu.SemaphoreType.DMA,
        ],
    )
    return k(positions_flat, pe_table)


def kernel(positions, pe_table):
    b, s = positions.shape
    out = _pe_lookup(positions.reshape(b * s), pe_table)
    return out.reshape(b, s, pe_table.shape[1])
